# Initial kernel scaffold; baseline (speedup 1.0000x reference)
import jax
import jax.numpy as jnp
from jax.experimental import pallas as pl


def kernel(x, edge_index, W_in0, W_ng0, b0, W_in1, W_ng1, b1, W_out, b_out):
    raise NotImplementedError("write your pallas kernel here")


# speedup vs baseline: 10.5856x; 10.5856x over previous
import functools

import jax
import jax.numpy as jnp
from jax import lax
from jax.experimental import pallas as pl
from jax.experimental.pallas import tpu as pltpu
from jax.experimental.pallas import tpu_sc as plsc

N = 10000
E = 320000
D = 128
NP = 10240
TRASH = NP
NSC = 2
NSUB = 16
EPT = E // (NSC * NSUB)
CH = 80
NCH = EPT // CH
RPT = NP // NSUB

_HIGH = jax.lax.Precision.HIGHEST


def _dot_t(a, b):
    return lax.dot_general(a, b, (((1,), (1,)), ((), ())),
                           precision=_HIGH, preferred_element_type=jnp.float32)


def _sc_deg_body(src_h, degp, srcb, ones_v, zer, deg_s, sem):
    c = lax.axis_index("c")
    s = lax.axis_index("s")

    def _fill(i, _):
        ones_v[pl.ds(i * 16, 16)] = jnp.ones((16,), jnp.float32)
        return 0
    lax.fori_loop(0, CH // 16, _fill, 0)

    def _zero(i, _):
        zer[pl.ds(i * 16, 16)] = jnp.zeros((16,), jnp.float32)
        return 0
    lax.fori_loop(0, RPT // 16, _zero, 0)
    pltpu.sync_copy(zer, deg_s.at[pl.ds(s * RPT, RPT)])
    plsc.subcore_barrier()

    base = (c * NSUB + s) * EPT

    def _chunk(k, _):
        off = pl.multiple_of(base + k * CH, 8)
        pltpu.sync_copy(src_h.at[pl.ds(off, CH)], srcb)
        pltpu.async_copy(ones_v, deg_s.at[srcb], sem, add=True).wait()
        return 0
    lax.fori_loop(0, NCH, _chunk, 0)

    plsc.subcore_barrier()
    pltpu.sync_copy(deg_s.at[pl.ds(s * RPT, RPT)],
                    degp.at[c, pl.ds(s * RPT, RPT)])


def _sc_agg_body(hp, src_h, dst_h, aggp, srcb, dstb, rows, agg_s, sem_g, sem_s):
    c = lax.axis_index("c")
    s = lax.axis_index("s")

    def _zrow(r, _):
        for j in range(D // 16):
            rows[r, pl.ds(j * 16, 16)] = jnp.zeros((16,), jnp.float32)
        return 0
    lax.fori_loop(0, CH, _zrow, 0)

    def _zcp(t, _):
        pltpu.sync_copy(rows, agg_s.at[pl.ds(s * RPT + t * CH, CH)])
        return 0
    lax.fori_loop(0, RPT // CH, _zcp, 0)

    @pl.when(s == 0)
    def _ztrash():
        pltpu.sync_copy(rows.at[pl.ds(0, 8)], agg_s.at[pl.ds(NP, 8)])

    plsc.subcore_barrier()

    base = (c * NSUB + s) * EPT

    def _chunk(k, _):
        off = pl.multiple_of(base + k * CH, 8)
        pltpu.sync_copy(src_h.at[pl.ds(off, CH)], srcb)
        pltpu.sync_copy(dst_h.at[pl.ds(off, CH)], dstb)
        for j in range(CH // 16):
            sv = srcb[pl.ds(j * 16, 16)]
            dv = dstb[pl.ds(j * 16, 16)]
            dstb[pl.ds(j * 16, 16)] = jnp.where(sv == dv, TRASH, dv)
        pltpu.async_copy(hp.at[srcb], rows, sem_g).wait()
        pltpu.async_copy(rows, agg_s.at[dstb], sem_s, add=True).wait()
        return 0
    lax.fori_loop(0, NCH, _chunk, 0)

    plsc.subcore_barrier()
    pltpu.sync_copy(agg_s.at[pl.ds(s * RPT, RPT)],
                    aggp.at[c, pl.ds(s * RPT, RPT)])


@functools.cache
def _sc_kernels():
    mesh = plsc.VectorSubcoreMesh(core_axis_name="c", subcore_axis_name="s",
                                  num_cores=NSC, num_subcores=NSUB)
    sc_deg = pl.kernel(
        _sc_deg_body,
        out_type=jax.ShapeDtypeStruct((NSC, NP), jnp.float32),
        mesh=mesh,
        scratch_types=[
            pltpu.VMEM((CH,), jnp.int32),
            pltpu.VMEM((CH,), jnp.float32),
            pltpu.VMEM((RPT,), jnp.float32),
            pltpu.VMEM_SHARED((NP,), jnp.float32),
            pltpu.SemaphoreType.DMA,
        ],
    )
    sc_agg = pl.kernel(
        _sc_agg_body,
        out_type=jax.ShapeDtypeStruct((NSC, NP, D), jnp.float32),
        mesh=mesh,
        scratch_types=[
            pltpu.VMEM((CH,), jnp.int32),
            pltpu.VMEM((CH,), jnp.int32),
            pltpu.VMEM((CH, D), jnp.float32),
            pltpu.VMEM_SHARED((NP + 8, D), jnp.float32),
            pltpu.SemaphoreType.DMA,
            pltpu.SemaphoreType.DMA,
        ],
    )
    return sc_deg, sc_agg


BR = 512


def _dinv_of(degr):
    deg = degr[:, 0:1] + degr[:, 1:2]
    return jnp.where(deg > 0, lax.rsqrt(deg), 0.0)


def _tc0_body(xr, wi, wg, degr, a0r, h0r):
    xb = xr[...]
    dinv = _dinv_of(degr[...])
    a0r[...] = _dot_t(xb, wi[...])
    h0r[...] = _dot_t(xb, wg[...]) * dinv


def _tc1_body(a0r, aggr, degr, br, wi, wg, a1r, h1r):
    dinv = _dinv_of(degr[...])
    agg = aggr[0] + aggr[1]
    y = jnp.maximum(a0r[...] + dinv * agg + br[...], 0.0)
    a1r[...] = _dot_t(y, wi[...])
    h1r[...] = _dot_t(y, wg[...]) * dinv


def _tc2_body(a1r, aggr, degr, br, wo, bor, outr):
    dinv = _dinv_of(degr[...])
    agg = aggr[0] + aggr[1]
    y = jnp.maximum(a1r[...] + dinv * agg + br[...], 0.0)
    outr[...] = _dot_t(y, wo[...]) + bor[...]


_row_spec = pl.BlockSpec((BR, D), lambda i: (i, 0))
_w_spec = pl.BlockSpec((D, D), lambda i: (0, 0))
_deg_spec = pl.BlockSpec((BR, 2), lambda i: (i, 0))
_agg_spec = pl.BlockSpec((NSC, BR, D), lambda i: (0, i, 0))
_b_spec = pl.BlockSpec((1, D), lambda i: (0, 0))
_rows_out = jax.ShapeDtypeStruct((NP, D), jnp.float32)

_tc0 = pl.pallas_call(
    _tc0_body, grid=(NP // BR,),
    in_specs=[_row_spec, _w_spec, _w_spec, _deg_spec],
    out_specs=[_row_spec, _row_spec],
    out_shape=[_rows_out, _rows_out],
)

_tc1 = pl.pallas_call(
    _tc1_body, grid=(NP // BR,),
    in_specs=[_row_spec, _agg_spec, _deg_spec, _b_spec, _w_spec, _w_spec],
    out_specs=[_row_spec, _row_spec],
    out_shape=[_rows_out, _rows_out],
)

_tc2 = pl.pallas_call(
    _tc2_body, grid=(NP // BR,),
    in_specs=[_row_spec, _agg_spec, _deg_spec, _b_spec, _w_spec, _b_spec],
    out_specs=_row_spec,
    out_shape=_rows_out,
)


@jax.jit
def kernel(x, edge_index, W_in0, W_ng0, b0, W_in1, W_ng1, b1, W_out, b_out):
    sc_deg, sc_agg = _sc_kernels()
    x_pad = jnp.zeros((NP, D), jnp.float32).at[:N].set(x)
    src = edge_index[0]
    dst = edge_index[1]
    degp = sc_deg(src)
    deg_t = degp.T
    a0, h0 = _tc0(x_pad, W_in0, W_ng0, deg_t)
    agg0 = sc_agg(h0, src, dst)
    a1, h1 = _tc1(a0, agg0, deg_t, b0.reshape(1, D), W_in1, W_ng1)
    agg1 = sc_agg(h1, src, dst)
    out = _tc2(a1, agg1, deg_t, b1.reshape(1, D), W_out, b_out.reshape(1, D))
    return out[:N]


# speedup vs baseline: 24.0086x; 2.2680x over previous
import functools

import jax
import jax.numpy as jnp
from jax import lax
from jax.experimental import pallas as pl
from jax.experimental.pallas import tpu as pltpu
from jax.experimental.pallas import tpu_sc as plsc

N = 10000
E = 320000
D = 128
NP = 10240
TRASH = NP
NSC = 2
NSUB = 16
EPT = E // (NSC * NSUB)
CH = 80
NCH = EPT // CH
RPT = NP // NSUB

_HIGH = jax.lax.Precision.HIGHEST


def _dot_t(a, b):
    return lax.dot_general(a, b, (((1,), (1,)), ((), ())),
                           precision=_HIGH, preferred_element_type=jnp.float32)


NSLOT = 4


def _sc_deg_body(src_h, degp, srcb_all, sb0, sb1, sb2, sb3,
                 ones_v, zer, deg_s, s0, s1, s2, s3):
    c = lax.axis_index("c")
    s = lax.axis_index("s")
    slots = (sb0, sb1, sb2, sb3)
    sems = (s0, s1, s2, s3)

    def _fill(i, _):
        ones_v[pl.ds(i * 16, 16)] = jnp.ones((16,), jnp.float32)
        return 0
    lax.fori_loop(0, CH // 16, _fill, 0)

    def _zero(i, _):
        zer[pl.ds(i * 16, 16)] = jnp.zeros((16,), jnp.float32)
        return 0
    lax.fori_loop(0, RPT // 16, _zero, 0)
    pltpu.sync_copy(zer, deg_s.at[pl.ds(s * RPT, RPT)])
    plsc.subcore_barrier()

    base = (c * NSUB + s) * EPT
    pltpu.sync_copy(src_h.at[pl.ds(pl.multiple_of(base, 8), EPT)], srcb_all)

    def _stage(r, k):
        for j in range(CH // 16):
            slots[r][pl.ds(j * 16, 16)] = srcb_all[pl.ds(k * CH + j * 16, 16)]

    def _fire(r):
        pltpu.async_copy(ones_v, deg_s.at[slots[r]], sems[r], add=True)

    def _swait(r):
        pltpu.make_async_copy(ones_v, deg_s.at[slots[r]], sems[r]).wait()

    for k in range(NSLOT):
        _stage(k, k)
        _fire(k)

    def _group(g, _):
        for r in range(NSLOT):
            k = g * NSLOT + r
            _swait(r)
            _stage(r, k)
            _fire(r)
        return 0
    lax.fori_loop(1, (NCH - 1) // NSLOT, _group, 0)

    for k in range(((NCH - 1) // NSLOT) * NSLOT, NCH):
        r = k % NSLOT
        _swait(r)
        _stage(r, k)
        _fire(r)
    for r in range(NSLOT):
        _swait(r)

    plsc.subcore_barrier()
    pltpu.sync_copy(deg_s.at[pl.ds(s * RPT, RPT)],
                    degp.at[c, pl.ds(s * RPT, RPT)])


NSLOT_A = 2


def _sc_agg_body(hp, src_h, dst_h, aggp, srcb_all, dst_all,
                 db0, db1, r0, r1, agg_s, g0, g1, t0, t1):
    c = lax.axis_index("c")
    s = lax.axis_index("s")
    dstb = (db0, db1)
    rows = (r0, r1)
    gsem = (g0, g1)
    ssem = (t0, t1)

    def _zrow(r, _):
        for j in range(D // 16):
            r0[r, pl.ds(j * 16, 16)] = jnp.zeros((16,), jnp.float32)
        return 0
    lax.fori_loop(0, CH, _zrow, 0)

    def _zcp(t, _):
        pltpu.sync_copy(r0, agg_s.at[pl.ds(s * RPT + t * CH, CH)])
        return 0
    lax.fori_loop(0, RPT // CH, _zcp, 0)

    @pl.when(s == 0)
    def _ztrash():
        pltpu.sync_copy(r0.at[pl.ds(0, 8)], agg_s.at[pl.ds(NP, 8)])

    plsc.subcore_barrier()

    base = pl.multiple_of((c * NSUB + s) * EPT, 8)
    pltpu.sync_copy(src_h.at[pl.ds(base, EPT)], srcb_all)
    pltpu.sync_copy(dst_h.at[pl.ds(base, EPT)], dst_all)

    def _stage(r, k):
        for j in range(CH // 16):
            sv = srcb_all[pl.ds(k * CH + j * 16, 16)]
            dv = dst_all[pl.ds(k * CH + j * 16, 16)]
            dstb[r][pl.ds(j * 16, 16)] = jnp.where(sv == dv, TRASH, dv)

    def _fire_gather(r, k):
        idx = srcb_all.at[pl.ds(k * CH, CH)]
        pltpu.async_copy(hp.at[idx], rows[r], gsem[r])

    def _wait_gather(r, k):
        idx = srcb_all.at[pl.ds(k * CH, CH)]
        pltpu.make_async_copy(hp.at[idx], rows[r], gsem[r]).wait()

    def _fire_scatter(r):
        pltpu.async_copy(rows[r], agg_s.at[dstb[r]], ssem[r], add=True)

    def _wait_scatter(r):
        pltpu.make_async_copy(rows[r], agg_s.at[dstb[r]], ssem[r]).wait()

    _stage(0, 0)
    _fire_gather(0, 0)
    for k in range(1, NSLOT_A):
        _stage(k, k)
        _fire_gather(k, k)
        _wait_gather(k - 1, k - 1)
        _fire_scatter(k - 1)

    def _group(g, _):
        for r in range(NSLOT_A):
            k = g * NSLOT_A + r
            _wait_scatter(r)
            _stage(r, k)
            _fire_gather(r, k)
            rp = (r + NSLOT_A - 1) % NSLOT_A
            _wait_gather(rp, k - 1)
            _fire_scatter(rp)
        return 0
    lax.fori_loop(1, (NCH - 1) // NSLOT_A, _group, 0)

    for k in range(((NCH - 1) // NSLOT_A) * NSLOT_A, NCH):
        r = k % NSLOT_A
        _wait_scatter(r)
        _stage(r, k)
        _fire_gather(r, k)
        rp = (r + NSLOT_A - 1) % NSLOT_A
        _wait_gather(rp, k - 1)
        _fire_scatter(rp)
    rl = (NCH - 1) % NSLOT_A
    _wait_gather(rl, NCH - 1)
    _fire_scatter(rl)
    for r in range(NSLOT_A):
        _wait_scatter(r)

    plsc.subcore_barrier()
    pltpu.sync_copy(agg_s.at[pl.ds(s * RPT, RPT)],
                    aggp.at[c, pl.ds(s * RPT, RPT)])


@functools.cache
def _sc_kernels():
    mesh = plsc.VectorSubcoreMesh(core_axis_name="c", subcore_axis_name="s",
                                  num_cores=NSC, num_subcores=NSUB)
    sc_deg = pl.kernel(
        _sc_deg_body,
        out_type=jax.ShapeDtypeStruct((NSC, NP), jnp.float32),
        mesh=mesh,
        scratch_types=(
            [pltpu.VMEM((EPT,), jnp.int32)]
            + [pltpu.VMEM((CH,), jnp.int32)] * NSLOT
            + [pltpu.VMEM((CH,), jnp.float32),
               pltpu.VMEM((RPT,), jnp.float32),
               pltpu.VMEM_SHARED((NP,), jnp.float32)]
            + [pltpu.SemaphoreType.DMA] * NSLOT
        ),
    )
    sc_agg = pl.kernel(
        _sc_agg_body,
        out_type=jax.ShapeDtypeStruct((NSC, NP, D), jnp.float32),
        mesh=mesh,
        scratch_types=(
            [pltpu.VMEM((EPT,), jnp.int32)] * 2
            + [pltpu.VMEM((CH,), jnp.int32)] * NSLOT_A
            + [pltpu.VMEM((CH, D), jnp.float32)] * NSLOT_A
            + [pltpu.VMEM_SHARED((NP + 8, D), jnp.float32)]
            + [pltpu.SemaphoreType.DMA] * (2 * NSLOT_A)
        ),
    )
    return sc_deg, sc_agg


BR = 512


def _dinv_of(degr):
    deg = degr[:, 0:1] + degr[:, 1:2]
    return jnp.where(deg > 0, lax.rsqrt(deg), 0.0)


def _tc0_body(xr, wi, wg, degr, a0r, h0r):
    xb = xr[...]
    dinv = _dinv_of(degr[...])
    a0r[...] = _dot_t(xb, wi[...])
    h0r[...] = _dot_t(xb, wg[...]) * dinv


def _tc1_body(a0r, aggr, degr, br, wi, wg, a1r, h1r):
    dinv = _dinv_of(degr[...])
    agg = aggr[0] + aggr[1]
    y = jnp.maximum(a0r[...] + dinv * agg + br[...], 0.0)
    a1r[...] = _dot_t(y, wi[...])
    h1r[...] = _dot_t(y, wg[...]) * dinv


def _tc2_body(a1r, aggr, degr, br, wo, bor, outr):
    dinv = _dinv_of(degr[...])
    agg = aggr[0] + aggr[1]
    y = jnp.maximum(a1r[...] + dinv * agg + br[...], 0.0)
    outr[...] = _dot_t(y, wo[...]) + bor[...]


_row_spec = pl.BlockSpec((BR, D), lambda i: (i, 0))
_w_spec = pl.BlockSpec((D, D), lambda i: (0, 0))
_deg_spec = pl.BlockSpec((BR, 2), lambda i: (i, 0))
_agg_spec = pl.BlockSpec((NSC, BR, D), lambda i: (0, i, 0))
_b_spec = pl.BlockSpec((1, D), lambda i: (0, 0))
_rows_out = jax.ShapeDtypeStruct((NP, D), jnp.float32)

_tc0 = pl.pallas_call(
    _tc0_body, grid=(NP // BR,),
    in_specs=[_row_spec, _w_spec, _w_spec, _deg_spec],
    out_specs=[_row_spec, _row_spec],
    out_shape=[_rows_out, _rows_out],
)

_tc1 = pl.pallas_call(
    _tc1_body, grid=(NP // BR,),
    in_specs=[_row_spec, _agg_spec, _deg_spec, _b_spec, _w_spec, _w_spec],
    out_specs=[_row_spec, _row_spec],
    out_shape=[_rows_out, _rows_out],
)

_tc2 = pl.pallas_call(
    _tc2_body, grid=(NP // BR,),
    in_specs=[_row_spec, _agg_spec, _deg_spec, _b_spec, _w_spec, _b_spec],
    out_specs=_row_spec,
    out_shape=_rows_out,
)


@jax.jit
def kernel(x, edge_index, W_in0, W_ng0, b0, W_in1, W_ng1, b1, W_out, b_out):
    sc_deg, sc_agg = _sc_kernels()
    x_pad = jnp.zeros((NP, D), jnp.float32).at[:N].set(x)
    src = edge_index[0]
    dst = edge_index[1]
    degp = sc_deg(src)
    deg_t = degp.T
    a0, h0 = _tc0(x_pad, W_in0, W_ng0, deg_t)
    agg0 = sc_agg(h0, src, dst)
    a1, h1 = _tc1(a0, agg0, deg_t, b0.reshape(1, D), W_in1, W_ng1)
    agg1 = sc_agg(h1, src, dst)
    out = _tc2(a1, agg1, deg_t, b1.reshape(1, D), W_out, b_out.reshape(1, D))
    return out[:N]
